# baseline (speedup 1.0000x reference)
import functools

import jax
import jax.numpy as jnp
from jax import lax
from jax.experimental import pallas as pl
from jax.experimental.pallas import tpu as pltpu
from jax.experimental.pallas import tpu_sc as plsc

V = 100000
E = 128
H = 256
OOV = 12
BB = 64
S = 200
OUTW = V + OOV
TW = 1024
NT = 98
BBLK = 8
NBB = BB // BBLK

_F32 = jnp.float32
_I32 = jnp.int32


def _sigmoid(x):
  return 1.0 / (1.0 + jnp.exp(-x))


def _dot_t(a, b, precision):
  return lax.dot_general(a, b, (((1,), (1,)), ((), ())),
                         preferred_element_type=_F32, precision=precision)


def _small_body(order_ref, inp_ref, emb_ref, wgt_ref, prev_ref, enc_ref,
                idx_ref, wih_ref, whh_ref, bih_ref, bhh_ref, wsw_ref, wsb_ref,
                wcw_ref, wcb_ref,
                state_out, mc_out, scs_out, pt_out, p_out, u_out):
  hi = jax.lax.Precision.HIGHEST
  enc = enc_ref[...]
  is_first = order_ref[0, 0] == 0
  enc_last = enc[:, S - 1, :]
  ps0 = _dot_t(enc_last, wsw_ref[...], hi) + wsb_ref[...]
  prev = jnp.where(is_first, ps0, prev_ref[...])
  wgt = jnp.where(is_first, jnp.zeros_like(wgt_ref[...]), wgt_ref[...])
  gru_x = jnp.concatenate([emb_ref[...], wgt], axis=1)
  gi = _dot_t(gru_x, wih_ref[...], hi) + bih_ref[...]
  gh = _dot_t(prev, whh_ref[...], hi) + bhh_ref[...]
  r = _sigmoid(gi[:, 0:H] + gh[:, 0:H])
  z = _sigmoid(gi[:, H:2 * H] + gh[:, H:2 * H])
  n = jnp.tanh(gi[:, 2 * H:3 * H] + r * gh[:, 2 * H:3 * H])
  state = (1.0 - z) * n + z * prev
  state_out[...] = state

  enc2 = enc.reshape(BBLK * S, 2 * H)
  sc2 = jnp.tanh(_dot_t(enc2, wcw_ref[...], hi) + wcb_ref[...])
  sc3 = sc2.reshape(BBLK, S, H)
  score_c = jnp.tanh(jnp.sum(sc3 * state[:, None, :], axis=2))
  idx = idx_ref[...]
  score_c = score_c + jnp.where(idx == 0, _F32(-1000.0), _F32(0.0))
  m = jnp.max(score_c, axis=1, keepdims=True)
  pt = jnp.exp(score_c - m)
  ssum = jnp.sum(pt, axis=1, keepdims=True)
  mc_out[...] = m
  scs_out[...] = ssum
  pt_out[...] = pt
  eq = (idx[:, :, None] == idx[:, None, :]).astype(_F32)
  p_out[...] = jnp.sum(eq * pt[:, None, :], axis=2)
  eqi = (idx == inp_ref[...]).astype(_F32)
  cnt = jnp.sum(eqi, axis=1, keepdims=True)
  w = jnp.where(cnt > 1.0, eqi / jnp.maximum(cnt, 1.0), eqi)
  pw = pt * w
  u_out[...] = jnp.sum(enc * pw[:, :, None], axis=1)


def _run_small(order_arr, inp2, emb, wgt2, prev_state, encoded, encidx,
               W_ih, W_hh, b_ih2, b_hh2, Ws_w, Ws_b2, Wc_w, Wc_b2):
  blk = lambda shape: pl.BlockSpec(shape, lambda i: (0,) * len(shape))
  bat = lambda *shape: pl.BlockSpec(shape, lambda i: (i,) + (0,) * (len(shape) - 1))
  return pl.pallas_call(
      _small_body,
      grid=(NBB,),
      in_specs=[
          blk((1, 1)),
          bat(BBLK, 1),
          bat(BBLK, E),
          bat(BBLK, 2 * H),
          bat(BBLK, H),
          bat(BBLK, S, 2 * H),
          bat(BBLK, S),
          blk((3 * H, E + 2 * H)),
          blk((3 * H, H)),
          blk((1, 3 * H)),
          blk((1, 3 * H)),
          blk((H, 2 * H)),
          blk((1, H)),
          blk((H, 2 * H)),
          blk((1, H)),
      ],
      out_specs=[
          bat(BBLK, H),
          bat(BBLK, 1),
          bat(BBLK, 1),
          bat(BBLK, S),
          bat(BBLK, S),
          bat(BBLK, 2 * H),
      ],
      out_shape=[
          jax.ShapeDtypeStruct((BB, H), _F32),
          jax.ShapeDtypeStruct((BB, 1), _F32),
          jax.ShapeDtypeStruct((BB, 1), _F32),
          jax.ShapeDtypeStruct((BB, S), _F32),
          jax.ShapeDtypeStruct((BB, S), _F32),
          jax.ShapeDtypeStruct((BB, 2 * H), _F32),
      ],
  )(order_arr, inp2, emb, wgt2, prev_state, encoded, encidx,
    W_ih, W_hh, b_ih2, b_hh2, Ws_w, Ws_b2, Wc_w, Wc_b2)


def _big_body(state_ref, wo_ref, wob_ref, mc_ref, scs_ref,
              out_ref, mg_out, sg_out,
              logits_scr, m_scr, s_scr, mz_scr, rz_scr):
  i = pl.program_id(0)

  @pl.when(i < NT)
  def _phase1():
    j = i
    t = lax.dot_general(state_ref[...].astype(jnp.bfloat16),
                        wo_ref[...].astype(jnp.bfloat16),
                        (((1,), (1,)), ((), ())),
                        preferred_element_type=_F32)
    t = t + wob_ref[...]
    cols = j * TW + lax.broadcasted_iota(_I32, (BB, TW), 1)
    tm = jnp.where(cols < V, t, _F32(-1e30))
    logits_scr[j] = tm
    rowmax = jnp.max(tm, axis=1, keepdims=True)
    rowsum = jnp.sum(jnp.exp(tm - rowmax), axis=1, keepdims=True)

    @pl.when(j == 0)
    def _():
      m_scr[...] = rowmax
      s_scr[...] = rowsum

    @pl.when(j > 0)
    def _():
      m_old = m_scr[...]
      m_new = jnp.maximum(m_old, rowmax)
      s_scr[...] = (s_scr[...] * jnp.exp(m_old - m_new)
                    + rowsum * jnp.exp(rowmax - m_new))
      m_scr[...] = m_new

  @pl.when(i == NT)
  def _stats():
    mg = m_scr[...]
    sg = s_scr[...]
    mg_out[...] = mg
    sg_out[...] = sg
    mm = jnp.maximum(mg, mc_ref[...])
    zz = (sg * jnp.exp(mg - mm) + scs_ref[...] * jnp.exp(mc_ref[...] - mm))
    mz_scr[...] = mm
    rz_scr[...] = 1.0 / zz

  @pl.when(i >= NT)
  def _phase2():
    j = i - NT
    lg = logits_scr[j]
    p = jnp.exp(lg - mz_scr[...]) * rz_scr[...]
    cols = j * TW + lax.broadcasted_iota(_I32, (BB, TW), 1)
    out_ref[...] = jnp.where(cols < V, p, _F32(1e-4))


def _run_big(state, Wo_w, Wo_b2, m_c, s_c):
  return pl.pallas_call(
      _big_body,
      grid=(2 * NT,),
      in_specs=[
          pl.BlockSpec((BB, H), lambda i: (0, 0)),
          pl.BlockSpec((TW, H), lambda i: (jnp.minimum(i, NT - 1), 0)),
          pl.BlockSpec((1, TW), lambda i: (0, jnp.minimum(i, NT - 1))),
          pl.BlockSpec((BB, 1), lambda i: (0, 0)),
          pl.BlockSpec((BB, 1), lambda i: (0, 0)),
      ],
      out_specs=[
          pl.BlockSpec((BB, TW), lambda i: (0, jnp.maximum(i - NT, 0))),
          pl.BlockSpec((BB, 1), lambda i: (0, 0)),
          pl.BlockSpec((BB, 1), lambda i: (0, 0)),
      ],
      out_shape=[
          jax.ShapeDtypeStruct((BB, OUTW), _F32),
          jax.ShapeDtypeStruct((BB, 1), _F32),
          jax.ShapeDtypeStruct((BB, 1), _F32),
      ],
      scratch_shapes=[
          pltpu.VMEM((NT, BB, TW), _F32),
          pltpu.VMEM((BB, 1), _F32),
          pltpu.VMEM((BB, 1), _F32),
          pltpu.VMEM((BB, 1), _F32),
          pltpu.VMEM((BB, 1), _F32),
      ],
  )(state, Wo_w, Wo_b2, m_c, s_c)


def _combine_body(mg_ref, sg_ref, mc_ref, scs_ref, pt_ref, p_ref, u_ref,
                  dup_ref, idx_ref, a_out, wout_out):
  mg = mg_ref[...]
  mc = mc_ref[...]
  mm = jnp.maximum(mg, mc)
  zz = sg_ref[...] * jnp.exp(mg - mm) + scs_ref[...] * jnp.exp(mc - mm)
  scale = jnp.exp(mc - mm) / zz
  pt = pt_ref[...]
  d = dup_ref[...]
  term1 = pt * (1.0 - d)
  t2v = d * pt * p_ref[...]
  idx = idx_ref[...]
  eq = (idx[:, :, None] == idx[:, None, :]).astype(_F32)
  s1 = jnp.sum(eq * term1[:, None, :], axis=2)
  s2 = jnp.sum(eq * t2v[:, None, :], axis=2)
  a_out[...] = scale * s1 + scale * scale * s2
  wout_out[...] = scale * u_ref[...]


def _run_combine(m_g, s_g, m_c, s_c, pt, P, U, isdup, encidx):
  bat = lambda *shape: pl.BlockSpec(shape, lambda i: (i,) + (0,) * (len(shape) - 1))
  return pl.pallas_call(
      _combine_body,
      grid=(NBB,),
      in_specs=[
          bat(BBLK, 1), bat(BBLK, 1), bat(BBLK, 1), bat(BBLK, 1),
          bat(BBLK, S), bat(BBLK, S), bat(BBLK, 2 * H),
          bat(BBLK, S), bat(BBLK, S),
      ],
      out_specs=[bat(BBLK, S), bat(BBLK, 2 * H)],
      out_shape=[
          jax.ShapeDtypeStruct((BB, S), _F32),
          jax.ShapeDtypeStruct((BB, 2 * H), _F32),
      ],
  )(m_g, s_g, m_c, s_c, pt, P, U, isdup, encidx)


_CHUNK = 1600
_NCHUNK = (BB * S) // _CHUNK


def _sc_prep_body(inp_hbm, table_hbm, encidx_hbm, zeros_hbm,
                  emb_hbm, dup_hbm,
                  idx64_v, rows_v, counts_v, ci_v, cd_v, sem):
  cid = lax.axis_index("c")
  sid = lax.axis_index("s")
  wid = sid * 2 + cid

  @pl.when(wid == 1)
  def _emb():
    pltpu.sync_copy(inp_hbm, idx64_v)
    pltpu.async_copy(table_hbm.at[idx64_v], rows_v, sem).wait()
    pltpu.sync_copy(rows_v, emb_hbm)

  @pl.when(wid == 0)
  def _counts():
    pltpu.sync_copy(zeros_hbm, counts_v)
    ones = jnp.ones((16,), _F32)

    def outer_add(c, _):
      pltpu.sync_copy(encidx_hbm.at[pl.ds(c * _CHUNK, _CHUNK)], ci_v)

      def inner_add(k, _):
        ii = ci_v[pl.ds(k * 16, 16)]
        plsc.addupdate_scatter(counts_v, [ii], ones)
        return 0

      lax.fori_loop(0, _CHUNK // 16, inner_add, 0)
      return 0

    lax.fori_loop(0, _NCHUNK, outer_add, 0)

    def outer_rd(c, _):
      pltpu.sync_copy(encidx_hbm.at[pl.ds(c * _CHUNK, _CHUNK)], ci_v)

      def inner_rd(k, _):
        ii = ci_v[pl.ds(k * 16, 16)]
        cv = plsc.load_gather(counts_v, [ii])
        cd_v[pl.ds(k * 16, 16)] = jnp.where(cv > 1.5, _F32(1.0), _F32(0.0))
        return 0

      lax.fori_loop(0, _CHUNK // 16, inner_rd, 0)
      pltpu.sync_copy(cd_v, dup_hbm.at[pl.ds(c * _CHUNK, _CHUNK)])
      return 0

    lax.fori_loop(0, _NCHUNK, outer_rd, 0)


def _run_sc_prep(inp_idx, embed_table, encidx_flat, zeros_v):
  mesh = plsc.VectorSubcoreMesh(core_axis_name="c", subcore_axis_name="s", num_cores=2, num_subcores=16)
  f = pl.kernel(
      _sc_prep_body,
      out_type=[
          jax.ShapeDtypeStruct((BB, E), _F32),
          jax.ShapeDtypeStruct((BB * S,), _F32),
      ],
      mesh=mesh,
      compiler_params=pltpu.CompilerParams(needs_layout_passes=False),
      scratch_types=[
          pltpu.VMEM((BB,), _I32),
          pltpu.VMEM((BB, E), _F32),
          pltpu.VMEM((V,), _F32),
          pltpu.VMEM((_CHUNK,), _I32),
          pltpu.VMEM((_CHUNK,), _F32),
          pltpu.SemaphoreType.DMA,
      ],
  )
  return f(inp_idx, embed_table, encidx_flat, zeros_v)


_SP = 208
_HW = 112


def _sc_scatter_body(idxp_hbm, ap_hbm, out_ref,
                     idx_v, a_v, pos_v, a2_v, base_v, val_v, sem):
  cid = lax.axis_index("c")
  sid = lax.axis_index("s")
  wid = sid * 2 + cid

  for r in range(2):
    b = wid * 2 + r
    pltpu.sync_copy(idxp_hbm.at[pl.ds(b * _SP, _SP)], idx_v.at[pl.ds(0, _SP)])
    pltpu.sync_copy(ap_hbm.at[pl.ds(b * _SP, _SP)], a_v.at[pl.ds(0, _SP)])
    idx_v[pl.ds(_SP, 16)] = idx_v[pl.ds(0, 16)]
    a_v[pl.ds(_SP, 16)] = a_v[pl.ds(0, 16)]
    boff = b * OUTW
    for h in range(2):
      for c in range(_HW // 16):
        src = idx_v[pl.ds(h * 104 + c * 16, 16)]
        pos_v[h, pl.ds(c * 16, 16)] = src + boff
        a2_v[h, pl.ds(c * 16, 16)] = a_v[pl.ds(h * 104 + c * 16, 16)]
    for h in range(2):
      pltpu.async_copy(out_ref.at[pos_v.at[h]], base_v.at[h], sem).wait()
    for h in range(2):
      for c in range(_HW // 16):
        val_v[h, pl.ds(c * 16, 16)] = (base_v[h, pl.ds(c * 16, 16)]
                                       + a2_v[h, pl.ds(c * 16, 16)])
    for h in range(2):
      pltpu.async_copy(val_v.at[h], out_ref.at[pos_v.at[h]], sem).wait()


def _run_sc_scatter(idxp, ap, out_ref):
  mesh = plsc.VectorSubcoreMesh(core_axis_name="c", subcore_axis_name="s", num_cores=2, num_subcores=16)
  f = pl.kernel(
      _sc_scatter_body,
      out_type=(),
      mesh=mesh,
      compiler_params=pltpu.CompilerParams(needs_layout_passes=False),
      scratch_types=[
          pltpu.VMEM((_SP + 16,), _I32),
          pltpu.VMEM((_SP + 16,), _F32),
          pltpu.VMEM((2, _HW), _I32),
          pltpu.VMEM((2, _HW), _F32),
          pltpu.VMEM((2, _HW), _F32),
          pltpu.VMEM((2, _HW), _F32),
          pltpu.SemaphoreType.DMA,
      ],
  )
  f(idxp, ap, out_ref)


def kernel(input_idx, encoded, encoded_idx, prev_state, weighted, order,
           embed_table, W_ih, W_hh, b_ih, b_hh, Ws_w, Ws_b, Wo_w, Wo_b,
           Wc_w, Wc_b):
  order_arr = jnp.asarray(order, _I32).reshape(1, 1)
  inp_i32 = input_idx.astype(_I32)
  inp2 = inp_i32.reshape(BB, 1)
  encidx = encoded_idx.astype(_I32)
  encidx_flat = encidx.reshape(-1)
  wgt2 = weighted.reshape(BB, 2 * H)
  Wo_b2 = Wo_b.reshape(1, V)
  b_ih2 = b_ih.reshape(1, 3 * H)
  b_hh2 = b_hh.reshape(1, 3 * H)
  Ws_b2 = Ws_b.reshape(1, H)
  Wc_b2 = Wc_b.reshape(1, H)
  zeros_v = jnp.zeros((V,), _F32)

  emb, dup_flat = _run_sc_prep(inp_i32, embed_table, encidx_flat, zeros_v)
  isdup = dup_flat.reshape(BB, S)

  state, m_c, s_c, pt, P, U = _run_small(
      order_arr, inp2, emb, wgt2, prev_state, encoded, encidx,
      W_ih, W_hh, b_ih2, b_hh2, Ws_w, Ws_b2, Wc_w, Wc_b2)

  out2d, m_g, s_g = _run_big(state, Wo_w, Wo_b2, m_c, s_c)

  A, wout = _run_combine(m_g, s_g, m_c, s_c, pt, P, U, isdup, encidx)

  idxp = jnp.pad(encidx, ((0, 0), (0, _SP - S)), mode="edge").reshape(-1)
  ap = jnp.pad(A, ((0, 0), (0, _SP - S)), mode="edge").reshape(-1)

  out_flat_ref = jax.new_ref(out2d.reshape(-1))
  _run_sc_scatter(idxp, ap, out_flat_ref)
  out = out_flat_ref[...].reshape(BB, 1, OUTW)

  return (out, state, wout.reshape(BB, 1, 2 * H))
